# baseline (speedup 1.0000x reference)
import functools

import jax
import jax.numpy as jnp
from jax import lax
from jax.experimental import pallas as pl
from jax.experimental.pallas import tpu as pltpu
from jax.experimental.pallas import tpu_sc as plsc

_E = 16
_TOPK = 2
_TM = 128
_TG = 512


def _gate_body(x_ref, gw_ref, w_ref, idx_ref, rnk_ref, tcnt_ref):
    xv = x_ref[...]
    logits = lax.dot_general(xv, gw_ref[...], (((1,), (1,)), ((), ())),
                             preferred_element_type=jnp.float32)
    s = jax.nn.sigmoid(logits)
    iota = lax.broadcasted_iota(jnp.int32, s.shape, 1)
    m1 = jnp.max(s, axis=1, keepdims=True)
    i1 = jnp.min(jnp.where(s >= m1, iota, _E), axis=1, keepdims=True)
    s2 = jnp.where(iota == i1, -1.0, s)
    m2 = jnp.max(s2, axis=1, keepdims=True)
    i2 = jnp.min(jnp.where(s2 >= m2, iota, _E), axis=1, keepdims=True)
    tot = m1 + m2
    w_ref[...] = jnp.concatenate([m1 / tot, m2 / tot], axis=1)
    idx_ref[...] = jnp.concatenate([i1, i2], axis=1)
    oh = ((iota == i1) | (iota == i2)).astype(jnp.float32)
    rr = lax.broadcasted_iota(jnp.int32, (_TG, _TG), 0)
    cc = lax.broadcasted_iota(jnp.int32, (_TG, _TG), 1)
    lt = (rr > cc).astype(jnp.float32)
    ranks = lax.dot_general(lt, oh, (((1,), (0,)), ((), ())),
                            preferred_element_type=jnp.float32)
    r1 = jnp.sum(jnp.where(iota == i1, ranks, 0.0), axis=1, keepdims=True)
    r2 = jnp.sum(jnp.where(iota == i2, ranks, 0.0), axis=1, keepdims=True)
    rnk_ref[...] = jnp.concatenate([r1, r2], axis=1).astype(jnp.int32)
    tcnt_ref[...] = jnp.sum(oh, axis=0).astype(jnp.int32).reshape(1, 1, _E)


def _gating(x_flat, gate_W):
    n, c = x_flat.shape
    nt = n // _TG
    return pl.pallas_call(
        _gate_body,
        grid=(nt,),
        in_specs=[
            pl.BlockSpec((_TG, c), lambda t: (t, 0)),
            pl.BlockSpec((_E, c), lambda t: (0, 0)),
        ],
        out_specs=[
            pl.BlockSpec((_TG, _TOPK), lambda t: (t, 0)),
            pl.BlockSpec((_TG, _TOPK), lambda t: (t, 0)),
            pl.BlockSpec((_TG, _TOPK), lambda t: (t, 0)),
            pl.BlockSpec((1, 1, _E), lambda t: (t, 0, 0)),
        ],
        out_shape=[
            jax.ShapeDtypeStruct((n, _TOPK), jnp.float32),
            jax.ShapeDtypeStruct((n, _TOPK), jnp.int32),
            jax.ShapeDtypeStruct((n, _TOPK), jnp.int32),
            jax.ShapeDtypeStruct((nt, 1, _E), jnp.int32),
        ],
    )(x_flat, gate_W)


def _sc_gather(table, idx):
    v, d = table.shape
    b = idx.shape[0]
    info = plsc.get_sparse_core_info()
    nw = info.num_cores * info.num_subcores
    assert b % nw == 0
    b_per_w = b // nw
    ch = 64 if b_per_w % 64 == 0 else b_per_w
    n_ch = b_per_w // ch
    mesh = plsc.VectorSubcoreMesh(core_axis_name="c", subcore_axis_name="s")

    @functools.partial(
        pl.kernel, mesh=mesh,
        out_type=jax.ShapeDtypeStruct((b, d), table.dtype),
        scratch_types=[
            pltpu.VMEM((ch,), jnp.int32),
            pltpu.VMEM((ch, d), table.dtype),
            pltpu.SemaphoreType.DMA,
        ],
    )
    def k(table_hbm, idx_hbm, out_hbm, idx_v, rows_v, sem):
        wid = lax.axis_index("s") * info.num_cores + lax.axis_index("c")
        base = wid * b_per_w

        def body(cc, carry):
            off = base + cc * ch
            pltpu.sync_copy(idx_hbm.at[pl.ds(off, ch)], idx_v)
            pltpu.async_copy(table_hbm.at[idx_v], rows_v, sem).wait()
            pltpu.sync_copy(rows_v, out_hbm.at[pl.ds(off, ch)])
            return carry

        lax.fori_loop(0, n_ch, body, 0)

    return k(table, idx)


def _ffn_body(te_ref, xs_ref, wg_ref, wd_ref, os_ref):
    h2 = wd_ref.shape[2]
    xv = xs_ref[...].astype(jnp.bfloat16)
    g = lax.dot_general(xv, wg_ref[0], (((1,), (1,)), ((), ())),
                        preferred_element_type=jnp.float32)
    y, gg = g[:, :h2], g[:, h2:]
    h = (y * (gg * jax.nn.sigmoid(gg))).astype(jnp.bfloat16)
    os_ref[...] = lax.dot_general(h, wd_ref[0], (((1,), (1,)), ((), ())),
                                  preferred_element_type=jnp.float32)


def _grouped_ffn(xs, expert_gate_W, expert_down_W, tile_expert):
    mp, c = xs.shape
    nt = mp // _TM
    h2 = expert_down_W.shape[2]
    grid_spec = pltpu.PrefetchScalarGridSpec(
        num_scalar_prefetch=1,
        grid=(nt,),
        in_specs=[
            pl.BlockSpec((_TM, c), lambda t, te: (t, 0)),
            pl.BlockSpec((1, 2 * h2, c), lambda t, te: (te[t], 0, 0)),
            pl.BlockSpec((1, c, h2), lambda t, te: (te[t], 0, 0)),
        ],
        out_specs=pl.BlockSpec((_TM, c), lambda t, te: (t, 0)),
    )
    return pl.pallas_call(
        _ffn_body,
        grid_spec=grid_spec,
        out_shape=jax.ShapeDtypeStruct((mp, c), jnp.float32),
    )(tile_expert, xs, expert_gate_W, expert_down_W)


def _shared_body(x_ref, wsg_ref, wsd_ref, r0_ref, r1_ref, w_ref, out_ref):
    hs = wsd_ref.shape[1]
    xv = x_ref[...].astype(jnp.bfloat16)
    g = lax.dot_general(xv, wsg_ref[...], (((1,), (1,)), ((), ())),
                        preferred_element_type=jnp.float32)
    y, gg = g[:, :hs], g[:, hs:]
    h = (y * (gg * jax.nn.sigmoid(gg))).astype(jnp.bfloat16)
    o = lax.dot_general(h, wsd_ref[...], (((1,), (1,)), ((), ())),
                        preferred_element_type=jnp.float32)
    wv = w_ref[...]
    out_ref[...] = o + wv[:, 0:1] * r0_ref[...] + wv[:, 1:2] * r1_ref[...]


def _shared_combine(x_flat, shared_gate_W, shared_down_W, routed, w2):
    n, c = x_flat.shape
    hs = shared_down_W.shape[1]
    tm = 256
    rt = n // tm
    return pl.pallas_call(
        _shared_body,
        grid=(rt,),
        in_specs=[
            pl.BlockSpec((tm, c), lambda t: (t, 0)),
            pl.BlockSpec((2 * hs, c), lambda t: (0, 0)),
            pl.BlockSpec((c, hs), lambda t: (0, 0)),
            pl.BlockSpec((tm, c), lambda t: (t, 0)),
            pl.BlockSpec((tm, c), lambda t: (t + rt, 0)),
            pl.BlockSpec((tm, _TOPK), lambda t: (t, 0)),
        ],
        out_specs=pl.BlockSpec((tm, c), lambda t: (t, 0)),
        out_shape=jax.ShapeDtypeStruct((n, c), jnp.float32),
    )(x_flat, shared_gate_W, shared_down_W, routed, routed, w2)


def kernel(x, gate_W, shared_gate_W, shared_down_W, expert_gate_W, expert_down_W):
    bs, ts, c = x.shape
    n = bs * ts
    x_flat = x.reshape(n, c)
    m = n * _TOPK
    mp = m + _E * _TM
    nt = mp // _TM

    w2, idx2, rnk2, tcnt = _gating(x_flat, gate_W)

    tcnt = tcnt.reshape(-1, _E)
    base_tile = jnp.cumsum(tcnt, axis=0) - tcnt
    cnt = jnp.sum(tcnt, axis=0)
    pad_cnt = ((cnt + _TM - 1) // _TM) * _TM
    pad_off = jnp.concatenate([jnp.zeros((1,), jnp.int32),
                               jnp.cumsum(pad_cnt).astype(jnp.int32)])
    base_tok = jnp.repeat(base_tile, _TG, axis=0)
    dst2 = (jnp.take(pad_off, idx2)
            + jnp.take_along_axis(base_tok, idx2, axis=1) + rnk2)
    dst = dst2.reshape(-1)
    pad_src = jnp.arange(mp, dtype=jnp.int32) % n
    gather_tok = pad_src.at[dst].set(jnp.arange(m, dtype=jnp.int32) // _TOPK)
    pos_all = jnp.concatenate([dst2[:, 0], dst2[:, 1]])
    tile_expert = jnp.clip(
        jnp.searchsorted(pad_off[1:], jnp.arange(nt, dtype=jnp.int32) * _TM,
                         side="right"), 0, _E - 1).astype(jnp.int32)

    xs = _sc_gather(x_flat, gather_tok)
    os_ = _grouped_ffn(xs, expert_gate_W.astype(jnp.bfloat16),
                       expert_down_W.astype(jnp.bfloat16), tile_expert)
    routed = _sc_gather(os_, pos_all)
    out = _shared_combine(x_flat, shared_gate_W.astype(jnp.bfloat16),
                          shared_down_W.astype(jnp.bfloat16), routed, w2)
    return out.reshape(bs, ts, c)


# speedup vs baseline: 1.3620x; 1.3620x over previous
import functools

import jax
import jax.numpy as jnp
from jax import lax
from jax.experimental import pallas as pl
from jax.experimental.pallas import tpu as pltpu
from jax.experimental.pallas import tpu_sc as plsc

_E = 16
_TOPK = 2
_TM = 128
_TG = 512


def _gate_body(x_ref, gw_ref, w_ref, idx_ref, rnk_ref, tcnt_ref):
    xv = x_ref[...]
    logits = lax.dot_general(xv, gw_ref[...], (((1,), (1,)), ((), ())),
                             preferred_element_type=jnp.float32)
    s = jax.nn.sigmoid(logits)
    iota = lax.broadcasted_iota(jnp.int32, s.shape, 1)
    m1 = jnp.max(s, axis=1, keepdims=True)
    i1 = jnp.min(jnp.where(s >= m1, iota, _E), axis=1, keepdims=True)
    s2 = jnp.where(iota == i1, -1.0, s)
    m2 = jnp.max(s2, axis=1, keepdims=True)
    i2 = jnp.min(jnp.where(s2 >= m2, iota, _E), axis=1, keepdims=True)
    tot = m1 + m2
    w_ref[...] = jnp.concatenate([m1 / tot, m2 / tot], axis=1)
    idx_ref[...] = jnp.concatenate([i1, i2], axis=1)
    oh = ((iota == i1) | (iota == i2)).astype(jnp.float32)
    rr = lax.broadcasted_iota(jnp.int32, (_TG, _TG), 0)
    cc = lax.broadcasted_iota(jnp.int32, (_TG, _TG), 1)
    lt = (rr > cc).astype(jnp.float32)
    ranks = lax.dot_general(lt, oh, (((1,), (0,)), ((), ())),
                            preferred_element_type=jnp.float32)
    r1 = jnp.sum(jnp.where(iota == i1, ranks, 0.0), axis=1, keepdims=True)
    r2 = jnp.sum(jnp.where(iota == i2, ranks, 0.0), axis=1, keepdims=True)
    rnk_ref[...] = jnp.concatenate([r1, r2], axis=1).astype(jnp.int32)
    tcnt_ref[...] = jnp.sum(oh, axis=0).astype(jnp.int32).reshape(1, 1, _E)


def _gating(x_flat, gate_W):
    n, c = x_flat.shape
    nt = n // _TG
    return pl.pallas_call(
        _gate_body,
        grid=(nt,),
        in_specs=[
            pl.BlockSpec((_TG, c), lambda t: (t, 0)),
            pl.BlockSpec((_E, c), lambda t: (0, 0)),
        ],
        out_specs=[
            pl.BlockSpec((_TG, _TOPK), lambda t: (t, 0)),
            pl.BlockSpec((_TG, _TOPK), lambda t: (t, 0)),
            pl.BlockSpec((_TG, _TOPK), lambda t: (t, 0)),
            pl.BlockSpec((1, 1, _E), lambda t: (t, 0, 0)),
        ],
        out_shape=[
            jax.ShapeDtypeStruct((n, _TOPK), jnp.float32),
            jax.ShapeDtypeStruct((n, _TOPK), jnp.int32),
            jax.ShapeDtypeStruct((n, _TOPK), jnp.int32),
            jax.ShapeDtypeStruct((nt, 1, _E), jnp.int32),
        ],
    )(x_flat, gate_W)


def _dst_body(idx_ref, rnk_ref, bt_ref, po_ref, d0_ref, d1_ref):
    idx = idx_ref[...]
    rnk = rnk_ref[...]
    base_row = bt_ref[0]
    poff = po_ref[...]

    def sel(tbl, ids):
        io = lax.broadcasted_iota(jnp.int32, (_TG, _E), 1)
        return jnp.sum(jnp.where(io == ids, tbl, 0), axis=1, keepdims=True)

    d0_ref[...] = (sel(poff, idx[:, 0:1]) + sel(base_row, idx[:, 0:1])
                   + rnk[:, 0:1])
    d1_ref[...] = (sel(poff, idx[:, 1:2]) + sel(base_row, idx[:, 1:2])
                   + rnk[:, 1:2])


def _dst_rows(idx2, rnk2, base_tile, pad_off16):
    n = idx2.shape[0]
    nt = n // _TG
    return pl.pallas_call(
        _dst_body,
        grid=(nt,),
        in_specs=[
            pl.BlockSpec((_TG, _TOPK), lambda t: (t, 0)),
            pl.BlockSpec((_TG, _TOPK), lambda t: (t, 0)),
            pl.BlockSpec((1, 1, _E), lambda t: (t, 0, 0)),
            pl.BlockSpec((1, _E), lambda t: (0, 0)),
        ],
        out_specs=[
            pl.BlockSpec((_TG, 1), lambda t: (t, 0)),
            pl.BlockSpec((_TG, 1), lambda t: (t, 0)),
        ],
        out_shape=[
            jax.ShapeDtypeStruct((n, 1), jnp.int32),
            jax.ShapeDtypeStruct((n, 1), jnp.int32),
        ],
    )(idx2, rnk2, base_tile.reshape(nt, 1, _E), pad_off16)


def _sc_scatter_pairs(x_flat, pos_all, mp):
    n, d = x_flat.shape
    info = plsc.get_sparse_core_info()
    nw = info.num_cores * info.num_subcores
    t_per_w = n // nw
    tc = 32 if t_per_w % 32 == 0 else t_per_w
    n_ch = t_per_w // tc
    mesh = plsc.VectorSubcoreMesh(core_axis_name="c", subcore_axis_name="s")

    @functools.partial(
        pl.kernel, mesh=mesh,
        out_type=jax.ShapeDtypeStruct((mp, d), jnp.float32),
        scratch_types=[
            pltpu.VMEM((tc,), jnp.int32),
            pltpu.VMEM((tc,), jnp.int32),
            pltpu.VMEM((tc, d), jnp.float32),
            pltpu.SemaphoreType.DMA,
        ],
    )
    def k(x_hbm, pos_hbm, xs_hbm, idx0_v, idx1_v, rows_v, sem):
        wid = lax.axis_index("s") * info.num_cores + lax.axis_index("c")
        base = wid * t_per_w

        def body(cc, carry):
            off = base + cc * tc
            pltpu.sync_copy(x_hbm.at[pl.ds(off, tc)], rows_v)
            pltpu.sync_copy(pos_hbm.at[pl.ds(off, tc)], idx0_v)
            pltpu.sync_copy(pos_hbm.at[pl.ds(n + off, tc)], idx1_v)
            c0 = pltpu.async_copy(rows_v, xs_hbm.at[idx0_v], sem)
            c1 = pltpu.async_copy(rows_v, xs_hbm.at[idx1_v], sem)
            c0.wait()
            c1.wait()
            return carry

        lax.fori_loop(0, n_ch, body, 0)

    return k(x_flat, pos_all)


def _sc_gather(table, idx):
    v, d = table.shape
    b = idx.shape[0]
    info = plsc.get_sparse_core_info()
    nw = info.num_cores * info.num_subcores
    assert b % nw == 0
    b_per_w = b // nw
    ch = 64 if b_per_w % 64 == 0 else b_per_w
    n_ch = b_per_w // ch
    mesh = plsc.VectorSubcoreMesh(core_axis_name="c", subcore_axis_name="s")

    @functools.partial(
        pl.kernel, mesh=mesh,
        out_type=jax.ShapeDtypeStruct((b, d), table.dtype),
        scratch_types=[
            pltpu.VMEM((ch,), jnp.int32),
            pltpu.VMEM((ch, d), table.dtype),
            pltpu.SemaphoreType.DMA,
        ],
    )
    def k(table_hbm, idx_hbm, out_hbm, idx_v, rows_v, sem):
        wid = lax.axis_index("s") * info.num_cores + lax.axis_index("c")
        base = wid * b_per_w

        def body(cc, carry):
            off = base + cc * ch
            pltpu.sync_copy(idx_hbm.at[pl.ds(off, ch)], idx_v)
            pltpu.async_copy(table_hbm.at[idx_v], rows_v, sem).wait()
            pltpu.sync_copy(rows_v, out_hbm.at[pl.ds(off, ch)])
            return carry

        lax.fori_loop(0, n_ch, body, 0)

    return k(table, idx)


def _ffn_body(te_ref, used_ref, xs_ref, wg_ref, wd_ref, os_ref):
    t = pl.program_id(0)

    @pl.when(used_ref[t] > 0)
    def _():
        h2 = wd_ref.shape[2]
        xv = xs_ref[...]
        g = lax.dot_general(xv, wg_ref[0], (((1,), (1,)), ((), ())),
                            preferred_element_type=jnp.float32)
        y, gg = g[:, :h2], g[:, h2:]
        h = y * (gg * jax.nn.sigmoid(gg))
        os_ref[...] = lax.dot_general(h, wd_ref[0], (((1,), (1,)), ((), ())),
                                      preferred_element_type=jnp.float32)


def _grouped_ffn(xs, expert_gate_W, expert_down_W, tile_expert, tile_used):
    mp, c = xs.shape
    nt = mp // _TM
    h2 = expert_down_W.shape[2]
    grid_spec = pltpu.PrefetchScalarGridSpec(
        num_scalar_prefetch=2,
        grid=(nt,),
        in_specs=[
            pl.BlockSpec((_TM, c), lambda t, te, u: (t, 0)),
            pl.BlockSpec((1, 2 * h2, c), lambda t, te, u: (te[t], 0, 0)),
            pl.BlockSpec((1, c, h2), lambda t, te, u: (te[t], 0, 0)),
        ],
        out_specs=pl.BlockSpec((_TM, c), lambda t, te, u: (t, 0)),
    )
    return pl.pallas_call(
        _ffn_body,
        grid_spec=grid_spec,
        out_shape=jax.ShapeDtypeStruct((mp, c), jnp.float32),
    )(tile_expert, tile_used, xs, expert_gate_W, expert_down_W)


def _shared_body(x_ref, wsg_ref, wsd_ref, r0_ref, r1_ref, w_ref, out_ref):
    hs = wsd_ref.shape[1]
    xv = x_ref[...]
    g = lax.dot_general(xv, wsg_ref[...], (((1,), (1,)), ((), ())),
                        preferred_element_type=jnp.float32)
    y, gg = g[:, :hs], g[:, hs:]
    h = y * (gg * jax.nn.sigmoid(gg))
    o = lax.dot_general(h, wsd_ref[...], (((1,), (1,)), ((), ())),
                        preferred_element_type=jnp.float32)
    wv = w_ref[...]
    out_ref[...] = o + wv[:, 0:1] * r0_ref[...] + wv[:, 1:2] * r1_ref[...]


def _shared_combine(x_flat, shared_gate_W, shared_down_W, routed, w2):
    n, c = x_flat.shape
    hs = shared_down_W.shape[1]
    tm = 256
    rt = n // tm
    return pl.pallas_call(
        _shared_body,
        grid=(rt,),
        in_specs=[
            pl.BlockSpec((tm, c), lambda t: (t, 0)),
            pl.BlockSpec((2 * hs, c), lambda t: (0, 0)),
            pl.BlockSpec((c, hs), lambda t: (0, 0)),
            pl.BlockSpec((tm, c), lambda t: (t, 0)),
            pl.BlockSpec((tm, c), lambda t: (t + rt, 0)),
            pl.BlockSpec((tm, _TOPK), lambda t: (t, 0)),
        ],
        out_specs=pl.BlockSpec((tm, c), lambda t: (t, 0)),
        out_shape=jax.ShapeDtypeStruct((n, c), jnp.float32),
    )(x_flat, shared_gate_W, shared_down_W, routed, routed, w2)


def kernel(x, gate_W, shared_gate_W, shared_down_W, expert_gate_W, expert_down_W):
    bs, ts, c = x.shape
    n = bs * ts
    x_flat = x.reshape(n, c)
    m = n * _TOPK
    mp = m + _E * _TM
    nt = mp // _TM

    w2, idx2, rnk2, tcnt = _gating(x_flat, gate_W)

    tcnt = tcnt.reshape(-1, _E)
    base_tile = jnp.cumsum(tcnt, axis=0) - tcnt
    cnt = jnp.sum(tcnt, axis=0)
    pad_cnt = ((cnt + _TM - 1) // _TM) * _TM
    pad_off = jnp.concatenate([jnp.zeros((1,), jnp.int32),
                               jnp.cumsum(pad_cnt).astype(jnp.int32)])
    tile_row = jnp.arange(nt, dtype=jnp.int32) * _TM
    tile_expert = jnp.clip(
        jnp.searchsorted(pad_off[1:], tile_row, side="right"),
        0, _E - 1).astype(jnp.int32)
    tile_used = (tile_row < pad_off[_E]).astype(jnp.int32)

    d0, d1 = _dst_rows(idx2, rnk2, base_tile, pad_off[:_E].reshape(1, _E))
    pos_all = jnp.concatenate([d0.reshape(n), d1.reshape(n)])

    xs = _sc_scatter_pairs(x_flat, pos_all, mp)
    os_ = _grouped_ffn(xs, expert_gate_W, expert_down_W, tile_expert, tile_used)
    routed = _sc_gather(os_, pos_all)
    out = _shared_combine(x_flat, shared_gate_W, shared_down_W, routed, w2)
    return out.reshape(bs, ts, c)


# speedup vs baseline: 1.3738x; 1.0087x over previous
import functools

import jax
import jax.numpy as jnp
from jax import lax
from jax.experimental import pallas as pl
from jax.experimental.pallas import tpu as pltpu
from jax.experimental.pallas import tpu_sc as plsc

_E = 16
_TOPK = 2
_TM = 128
_TG = 512


def _gate_body(x_ref, gw_ref, w_ref, idx_ref, rnk_ref, tcnt_ref):
    xv = x_ref[...]
    logits = lax.dot_general(xv, gw_ref[...], (((1,), (1,)), ((), ())),
                             preferred_element_type=jnp.float32)
    s = jax.nn.sigmoid(logits)
    iota = lax.broadcasted_iota(jnp.int32, s.shape, 1)
    m1 = jnp.max(s, axis=1, keepdims=True)
    i1 = jnp.min(jnp.where(s >= m1, iota, _E), axis=1, keepdims=True)
    s2 = jnp.where(iota == i1, -1.0, s)
    m2 = jnp.max(s2, axis=1, keepdims=True)
    i2 = jnp.min(jnp.where(s2 >= m2, iota, _E), axis=1, keepdims=True)
    tot = m1 + m2
    w_ref[...] = jnp.concatenate([m1 / tot, m2 / tot], axis=1)
    idx_ref[...] = jnp.concatenate([i1, i2], axis=1)
    oh = ((iota == i1) | (iota == i2)).astype(jnp.float32)
    rr = lax.broadcasted_iota(jnp.int32, (_TG, _TG), 0)
    cc = lax.broadcasted_iota(jnp.int32, (_TG, _TG), 1)
    lt = (rr > cc).astype(jnp.float32)
    ranks = lax.dot_general(lt, oh, (((1,), (0,)), ((), ())),
                            preferred_element_type=jnp.float32)
    r1 = jnp.sum(jnp.where(iota == i1, ranks, 0.0), axis=1, keepdims=True)
    r2 = jnp.sum(jnp.where(iota == i2, ranks, 0.0), axis=1, keepdims=True)
    rnk_ref[...] = jnp.concatenate([r1, r2], axis=1).astype(jnp.int32)
    tcnt_ref[...] = jnp.sum(oh, axis=0).astype(jnp.int32).reshape(1, 1, _E)


def _gating(x_flat, gate_W):
    n, c = x_flat.shape
    nt = n // _TG
    return pl.pallas_call(
        _gate_body,
        grid=(nt,),
        in_specs=[
            pl.BlockSpec((_TG, c), lambda t: (t, 0)),
            pl.BlockSpec((_E, c), lambda t: (0, 0)),
        ],
        out_specs=[
            pl.BlockSpec((_TG, _TOPK), lambda t: (t, 0)),
            pl.BlockSpec((_TG, _TOPK), lambda t: (t, 0)),
            pl.BlockSpec((_TG, _TOPK), lambda t: (t, 0)),
            pl.BlockSpec((1, 1, _E), lambda t: (t, 0, 0)),
        ],
        out_shape=[
            jax.ShapeDtypeStruct((n, _TOPK), jnp.float32),
            jax.ShapeDtypeStruct((n, _TOPK), jnp.int32),
            jax.ShapeDtypeStruct((n, _TOPK), jnp.int32),
            jax.ShapeDtypeStruct((nt, 1, _E), jnp.int32),
        ],
    )(x_flat, gate_W)


def _dst_body(idx_ref, rnk_ref, bt_ref, po_ref, d0_ref, d1_ref):
    idx = idx_ref[...]
    rnk = rnk_ref[...]
    base_row = bt_ref[0]
    poff = po_ref[...]

    def sel(tbl, ids):
        io = lax.broadcasted_iota(jnp.int32, (_TG, _E), 1)
        return jnp.sum(jnp.where(io == ids, tbl, 0), axis=1, keepdims=True)

    d0_ref[...] = (sel(poff, idx[:, 0:1]) + sel(base_row, idx[:, 0:1])
                   + rnk[:, 0:1])
    d1_ref[...] = (sel(poff, idx[:, 1:2]) + sel(base_row, idx[:, 1:2])
                   + rnk[:, 1:2])


def _dst_rows(idx2, rnk2, base_tile, pad_off16):
    n = idx2.shape[0]
    nt = n // _TG
    return pl.pallas_call(
        _dst_body,
        grid=(nt,),
        in_specs=[
            pl.BlockSpec((_TG, _TOPK), lambda t: (t, 0)),
            pl.BlockSpec((_TG, _TOPK), lambda t: (t, 0)),
            pl.BlockSpec((1, 1, _E), lambda t: (t, 0, 0)),
            pl.BlockSpec((1, _E), lambda t: (0, 0)),
        ],
        out_specs=[
            pl.BlockSpec((_TG, 1), lambda t: (t, 0)),
            pl.BlockSpec((_TG, 1), lambda t: (t, 0)),
        ],
        out_shape=[
            jax.ShapeDtypeStruct((n, 1), jnp.int32),
            jax.ShapeDtypeStruct((n, 1), jnp.int32),
        ],
    )(idx2, rnk2, base_tile.reshape(nt, 1, _E), pad_off16)


def _sc_scatter_pairs(x_flat, pos_all, mp):
    n, d = x_flat.shape
    info = plsc.get_sparse_core_info()
    nw = info.num_cores * info.num_subcores
    t_per_w = n // nw
    tc = 32 if t_per_w % 32 == 0 else t_per_w
    n_ch = t_per_w // tc
    mesh = plsc.VectorSubcoreMesh(core_axis_name="c", subcore_axis_name="s")

    @functools.partial(
        pl.kernel, mesh=mesh,
        out_type=jax.ShapeDtypeStruct((mp, d), jnp.float32),
        scratch_types=[
            pltpu.VMEM((tc,), jnp.int32),
            pltpu.VMEM((tc,), jnp.int32),
            pltpu.VMEM((tc, d), jnp.float32),
            pltpu.SemaphoreType.DMA,
        ],
    )
    def k(x_hbm, pos_hbm, xs_hbm, idx0_v, idx1_v, rows_v, sem):
        wid = lax.axis_index("s") * info.num_cores + lax.axis_index("c")
        base = wid * t_per_w

        def body(cc, carry):
            off = base + cc * tc
            pltpu.sync_copy(x_hbm.at[pl.ds(off, tc)], rows_v)
            pltpu.sync_copy(pos_hbm.at[pl.ds(off, tc)], idx0_v)
            pltpu.sync_copy(pos_hbm.at[pl.ds(n + off, tc)], idx1_v)
            c0 = pltpu.async_copy(rows_v, xs_hbm.at[idx0_v], sem)
            c1 = pltpu.async_copy(rows_v, xs_hbm.at[idx1_v], sem)
            c0.wait()
            c1.wait()
            return carry

        lax.fori_loop(0, n_ch, body, 0)

    return k(x_flat, pos_all)


def _sc_gather(table, idx):
    v, d = table.shape
    b = idx.shape[0]
    info = plsc.get_sparse_core_info()
    nw = info.num_cores * info.num_subcores
    assert b % nw == 0
    b_per_w = b // nw
    ch = 64 if b_per_w % 64 == 0 else b_per_w
    n_ch = b_per_w // ch
    mesh = plsc.VectorSubcoreMesh(core_axis_name="c", subcore_axis_name="s")

    @functools.partial(
        pl.kernel, mesh=mesh,
        out_type=jax.ShapeDtypeStruct((b, d), table.dtype),
        scratch_types=[
            pltpu.VMEM((ch,), jnp.int32),
            pltpu.VMEM((ch, d), table.dtype),
            pltpu.SemaphoreType.DMA,
        ],
    )
    def k(table_hbm, idx_hbm, out_hbm, idx_v, rows_v, sem):
        wid = lax.axis_index("s") * info.num_cores + lax.axis_index("c")
        base = wid * b_per_w

        def body(cc, carry):
            off = base + cc * ch
            pltpu.sync_copy(idx_hbm.at[pl.ds(off, ch)], idx_v)
            pltpu.async_copy(table_hbm.at[idx_v], rows_v, sem).wait()
            pltpu.sync_copy(rows_v, out_hbm.at[pl.ds(off, ch)])
            return carry

        lax.fori_loop(0, n_ch, body, 0)

    return k(table, idx)


def _ffn_body(te_ref, used_ref, xs_ref, wg_ref, wd_ref, os_ref):
    t = pl.program_id(0)

    @pl.when(used_ref[t] > 0)
    def _():
        h2 = wd_ref.shape[2]
        xv = xs_ref[...]
        g = lax.dot_general(xv, wg_ref[0], (((1,), (1,)), ((), ())),
                            preferred_element_type=jnp.float32)
        y, gg = g[:, :h2], g[:, h2:]
        h = y * (gg * jax.nn.sigmoid(gg))
        os_ref[...] = lax.dot_general(h, wd_ref[0], (((1,), (1,)), ((), ())),
                                      preferred_element_type=jnp.float32)


def _grouped_ffn(xs, expert_gate_W, expert_down_W, tile_expert, tile_used):
    mp, c = xs.shape
    nt = mp // _TM
    h2 = expert_down_W.shape[2]
    grid_spec = pltpu.PrefetchScalarGridSpec(
        num_scalar_prefetch=2,
        grid=(nt,),
        in_specs=[
            pl.BlockSpec((_TM, c), lambda t, te, u: (t, 0)),
            pl.BlockSpec((1, 2 * h2, c), lambda t, te, u: (te[t], 0, 0)),
            pl.BlockSpec((1, c, h2), lambda t, te, u: (te[t], 0, 0)),
        ],
        out_specs=pl.BlockSpec((_TM, c), lambda t, te, u: (t, 0)),
    )
    return pl.pallas_call(
        _ffn_body,
        grid_spec=grid_spec,
        out_shape=jax.ShapeDtypeStruct((mp, c), jnp.float32),
    )(tile_expert, tile_used, xs, expert_gate_W, expert_down_W)


def _shared_body(x_ref, wsg_ref, wsd_ref, out_ref):
    hs = wsd_ref.shape[1]
    xv = x_ref[...]
    g = lax.dot_general(xv, wsg_ref[...], (((1,), (1,)), ((), ())),
                        preferred_element_type=jnp.float32)
    y, gg = g[:, :hs], g[:, hs:]
    h = y * (gg * jax.nn.sigmoid(gg))
    out_ref[...] = lax.dot_general(h, wsd_ref[...], (((1,), (1,)), ((), ())),
                                   preferred_element_type=jnp.float32)


def _shared_ffn(x_flat, shared_gate_W, shared_down_W):
    n, c = x_flat.shape
    hs = shared_down_W.shape[1]
    tm = 256
    rt = n // tm
    return pl.pallas_call(
        _shared_body,
        grid=(rt,),
        in_specs=[
            pl.BlockSpec((tm, c), lambda t: (t, 0)),
            pl.BlockSpec((2 * hs, c), lambda t: (0, 0)),
            pl.BlockSpec((c, hs), lambda t: (0, 0)),
        ],
        out_specs=pl.BlockSpec((tm, c), lambda t: (t, 0)),
        out_shape=jax.ShapeDtypeStruct((n, c), jnp.float32),
    )(x_flat, shared_gate_W, shared_down_W)


def _combine_body(sh_ref, r0_ref, r1_ref, w_ref, out_ref):
    wv = w_ref[...]
    out_ref[...] = (sh_ref[...] + wv[:, 0:1] * r0_ref[...]
                    + wv[:, 1:2] * r1_ref[...])


def _combine(shared_out, routed, w2):
    n, c = shared_out.shape
    tm = 512
    rt = n // tm
    return pl.pallas_call(
        _combine_body,
        grid=(rt,),
        in_specs=[
            pl.BlockSpec((tm, c), lambda t: (t, 0)),
            pl.BlockSpec((tm, c), lambda t: (t, 0)),
            pl.BlockSpec((tm, c), lambda t: (t + rt, 0)),
            pl.BlockSpec((tm, _TOPK), lambda t: (t, 0)),
        ],
        out_specs=pl.BlockSpec((tm, c), lambda t: (t, 0)),
        out_shape=jax.ShapeDtypeStruct((n, c), jnp.float32),
    )(shared_out, routed, routed, w2)


def kernel(x, gate_W, shared_gate_W, shared_down_W, expert_gate_W, expert_down_W):
    bs, ts, c = x.shape
    n = bs * ts
    x_flat = x.reshape(n, c)
    m = n * _TOPK
    mp = m + _E * _TM
    nt = mp // _TM

    w2, idx2, rnk2, tcnt = _gating(x_flat, gate_W)

    tcnt = tcnt.reshape(-1, _E)
    base_tile = jnp.cumsum(tcnt, axis=0) - tcnt
    cnt = jnp.sum(tcnt, axis=0)
    pad_cnt = ((cnt + _TM - 1) // _TM) * _TM
    pad_off = jnp.concatenate([jnp.zeros((1,), jnp.int32),
                               jnp.cumsum(pad_cnt).astype(jnp.int32)])
    tile_row = jnp.arange(nt, dtype=jnp.int32) * _TM
    tile_expert = jnp.clip(
        jnp.searchsorted(pad_off[1:], tile_row, side="right"),
        0, _E - 1).astype(jnp.int32)
    tile_used = (tile_row < pad_off[_E]).astype(jnp.int32)

    d0, d1 = _dst_rows(idx2, rnk2, base_tile, pad_off[:_E].reshape(1, _E))
    pos_all = jnp.concatenate([d0.reshape(n), d1.reshape(n)])

    xs = _sc_scatter_pairs(x_flat, pos_all, mp)
    shared_out = _shared_ffn(x_flat, shared_gate_W, shared_down_W)
    os_ = _grouped_ffn(xs, expert_gate_W, expert_down_W, tile_expert, tile_used)
    routed = _sc_gather(os_, pos_all)
    out = _combine(shared_out, routed, w2)
    return out.reshape(bs, ts, c)


# speedup vs baseline: 1.7570x; 1.2789x over previous
import functools

import jax
import jax.numpy as jnp
from jax import lax
from jax.experimental import pallas as pl
from jax.experimental.pallas import tpu as pltpu
from jax.experimental.pallas import tpu_sc as plsc

_E = 16
_TOPK = 2
_TM = 256
_TG = 512


def _gate_body(x_ref, gw_ref, w_ref, idx_ref, rnk_ref, tcnt_ref):
    xv = x_ref[...]
    logits = lax.dot_general(xv, gw_ref[...], (((1,), (1,)), ((), ())),
                             preferred_element_type=jnp.float32)
    s = jax.nn.sigmoid(logits)
    iota = lax.broadcasted_iota(jnp.int32, s.shape, 1)
    m1 = jnp.max(s, axis=1, keepdims=True)
    i1 = jnp.min(jnp.where(s >= m1, iota, _E), axis=1, keepdims=True)
    s2 = jnp.where(iota == i1, -1.0, s)
    m2 = jnp.max(s2, axis=1, keepdims=True)
    i2 = jnp.min(jnp.where(s2 >= m2, iota, _E), axis=1, keepdims=True)
    tot = m1 + m2
    w_ref[...] = jnp.concatenate([m1 / tot, m2 / tot], axis=1)
    idx_ref[...] = jnp.concatenate([i1, i2], axis=1)
    oh = ((iota == i1) | (iota == i2)).astype(jnp.float32)
    rr = lax.broadcasted_iota(jnp.int32, (_TG, _TG), 0)
    cc = lax.broadcasted_iota(jnp.int32, (_TG, _TG), 1)
    lt = (rr > cc).astype(jnp.float32)
    ranks = lax.dot_general(lt, oh, (((1,), (0,)), ((), ())),
                            preferred_element_type=jnp.float32)
    r1 = jnp.sum(jnp.where(iota == i1, ranks, 0.0), axis=1, keepdims=True)
    r2 = jnp.sum(jnp.where(iota == i2, ranks, 0.0), axis=1, keepdims=True)
    rnk_ref[...] = jnp.concatenate([r1, r2], axis=1).astype(jnp.int32)
    tcnt_ref[...] = jnp.sum(oh, axis=0).astype(jnp.int32).reshape(1, 1, _E)


def _gating(x_flat, gate_W):
    n, c = x_flat.shape
    nt = n // _TG
    return pl.pallas_call(
        _gate_body,
        grid=(nt,),
        in_specs=[
            pl.BlockSpec((_TG, c), lambda t: (t, 0)),
            pl.BlockSpec((_E, c), lambda t: (0, 0)),
        ],
        out_specs=[
            pl.BlockSpec((_TG, _TOPK), lambda t: (t, 0)),
            pl.BlockSpec((_TG, _TOPK), lambda t: (t, 0)),
            pl.BlockSpec((_TG, _TOPK), lambda t: (t, 0)),
            pl.BlockSpec((1, 1, _E), lambda t: (t, 0, 0)),
        ],
        out_shape=[
            jax.ShapeDtypeStruct((n, _TOPK), jnp.float32),
            jax.ShapeDtypeStruct((n, _TOPK), jnp.int32),
            jax.ShapeDtypeStruct((n, _TOPK), jnp.int32),
            jax.ShapeDtypeStruct((nt, 1, _E), jnp.int32),
        ],
    )(x_flat, gate_W)


def _dst_body(idx_ref, rnk_ref, bt_ref, po_ref, d0_ref, d1_ref):
    idx = idx_ref[...]
    rnk = rnk_ref[...]
    base_row = bt_ref[0]
    poff = po_ref[...]

    def sel(tbl, ids):
        io = lax.broadcasted_iota(jnp.int32, (_TG, _E), 1)
        return jnp.sum(jnp.where(io == ids, tbl, 0), axis=1, keepdims=True)

    d0_ref[...] = (sel(poff, idx[:, 0:1]) + sel(base_row, idx[:, 0:1])
                   + rnk[:, 0:1])
    d1_ref[...] = (sel(poff, idx[:, 1:2]) + sel(base_row, idx[:, 1:2])
                   + rnk[:, 1:2])


def _dst_rows(idx2, rnk2, base_tile, pad_off16):
    n = idx2.shape[0]
    nt = n // _TG
    return pl.pallas_call(
        _dst_body,
        grid=(nt,),
        in_specs=[
            pl.BlockSpec((_TG, _TOPK), lambda t: (t, 0)),
            pl.BlockSpec((_TG, _TOPK), lambda t: (t, 0)),
            pl.BlockSpec((1, 1, _E), lambda t: (t, 0, 0)),
            pl.BlockSpec((1, _E), lambda t: (0, 0)),
        ],
        out_specs=[
            pl.BlockSpec((_TG, 1), lambda t: (t, 0)),
            pl.BlockSpec((_TG, 1), lambda t: (t, 0)),
        ],
        out_shape=[
            jax.ShapeDtypeStruct((n, 1), jnp.int32),
            jax.ShapeDtypeStruct((n, 1), jnp.int32),
        ],
    )(idx2, rnk2, base_tile.reshape(nt, 1, _E), pad_off16)


def _sc_scatter_pairs(x_flat, pos_all, mp):
    n, d = x_flat.shape
    info = plsc.get_sparse_core_info()
    nw = info.num_cores * info.num_subcores
    t_per_w = n // nw
    tc = 32 if t_per_w % 32 == 0 else t_per_w
    n_ch = t_per_w // tc
    mesh = plsc.VectorSubcoreMesh(core_axis_name="c", subcore_axis_name="s")

    @functools.partial(
        pl.kernel, mesh=mesh,
        out_type=jax.ShapeDtypeStruct((mp, d), jnp.float32),
        scratch_types=[
            pltpu.VMEM((tc,), jnp.int32),
            pltpu.VMEM((tc,), jnp.int32),
            pltpu.VMEM((tc, d), jnp.float32),
            pltpu.SemaphoreType.DMA,
        ],
    )
    def k(x_hbm, pos_hbm, xs_hbm, idx0_v, idx1_v, rows_v, sem):
        wid = lax.axis_index("s") * info.num_cores + lax.axis_index("c")
        base = wid * t_per_w

        def body(cc, carry):
            off = base + cc * tc
            pltpu.sync_copy(x_hbm.at[pl.ds(off, tc)], rows_v)
            pltpu.sync_copy(pos_hbm.at[pl.ds(off, tc)], idx0_v)
            pltpu.sync_copy(pos_hbm.at[pl.ds(n + off, tc)], idx1_v)
            c0 = pltpu.async_copy(rows_v, xs_hbm.at[idx0_v], sem)
            c1 = pltpu.async_copy(rows_v, xs_hbm.at[idx1_v], sem)
            c0.wait()
            c1.wait()
            return carry

        lax.fori_loop(0, n_ch, body, 0)

    return k(x_flat, pos_all)


def _sc_gather(table, idx):
    v, d = table.shape
    b = idx.shape[0]
    info = plsc.get_sparse_core_info()
    nw = info.num_cores * info.num_subcores
    assert b % nw == 0
    b_per_w = b // nw
    ch = 64 if b_per_w % 64 == 0 else b_per_w
    n_ch = b_per_w // ch
    mesh = plsc.VectorSubcoreMesh(core_axis_name="c", subcore_axis_name="s")

    @functools.partial(
        pl.kernel, mesh=mesh,
        out_type=jax.ShapeDtypeStruct((b, d), table.dtype),
        scratch_types=[
            pltpu.VMEM((ch,), jnp.int32),
            pltpu.VMEM((ch, d), table.dtype),
            pltpu.SemaphoreType.DMA,
        ],
    )
    def k(table_hbm, idx_hbm, out_hbm, idx_v, rows_v, sem):
        wid = lax.axis_index("s") * info.num_cores + lax.axis_index("c")
        base = wid * b_per_w

        def body(cc, carry):
            off = base + cc * ch
            pltpu.sync_copy(idx_hbm.at[pl.ds(off, ch)], idx_v)
            pltpu.async_copy(table_hbm.at[idx_v], rows_v, sem).wait()
            pltpu.sync_copy(rows_v, out_hbm.at[pl.ds(off, ch)])
            return carry

        lax.fori_loop(0, n_ch, body, 0)

    return k(table, idx)


def _ffn_body(te_ref, used_ref, xs_ref, wg_ref, wd_ref, os_ref):
    t = pl.program_id(0)

    @pl.when(used_ref[t] > 0)
    def _():
        h2 = wd_ref.shape[2]
        xv = xs_ref[...]
        g = lax.dot_general(xv, wg_ref[0], (((1,), (1,)), ((), ())),
                            preferred_element_type=jnp.float32)
        y, gg = g[:, :h2], g[:, h2:]
        h = y * (gg * jax.nn.sigmoid(gg))
        os_ref[...] = lax.dot_general(h, wd_ref[0], (((1,), (1,)), ((), ())),
                                      preferred_element_type=jnp.float32)


def _grouped_ffn(xs, expert_gate_W, expert_down_W, tile_expert, tile_used):
    mp, c = xs.shape
    nt = mp // _TM
    h2 = expert_down_W.shape[2]
    grid_spec = pltpu.PrefetchScalarGridSpec(
        num_scalar_prefetch=2,
        grid=(nt,),
        in_specs=[
            pl.BlockSpec((_TM, c), lambda t, te, u: (t, 0)),
            pl.BlockSpec((1, 2 * h2, c), lambda t, te, u: (te[t], 0, 0)),
            pl.BlockSpec((1, c, h2), lambda t, te, u: (te[t], 0, 0)),
        ],
        out_specs=pl.BlockSpec((_TM, c), lambda t, te, u: (t, 0)),
    )
    return pl.pallas_call(
        _ffn_body,
        grid_spec=grid_spec,
        out_shape=jax.ShapeDtypeStruct((mp, c), jnp.float32),
    )(tile_expert, tile_used, xs, expert_gate_W, expert_down_W)


def _shared_body(x_ref, wsg_ref, wsd_ref, out_ref):
    hs = wsd_ref.shape[1]
    xv = x_ref[...]
    g = lax.dot_general(xv, wsg_ref[...], (((1,), (1,)), ((), ())),
                        preferred_element_type=jnp.float32)
    y, gg = g[:, :hs], g[:, hs:]
    h = y * (gg * jax.nn.sigmoid(gg))
    out_ref[...] = lax.dot_general(h, wsd_ref[...], (((1,), (1,)), ((), ())),
                                   preferred_element_type=jnp.float32)


def _shared_ffn(x_flat, shared_gate_W, shared_down_W):
    n, c = x_flat.shape
    hs = shared_down_W.shape[1]
    tm = 256
    rt = n // tm
    return pl.pallas_call(
        _shared_body,
        grid=(rt,),
        in_specs=[
            pl.BlockSpec((tm, c), lambda t: (t, 0)),
            pl.BlockSpec((2 * hs, c), lambda t: (0, 0)),
            pl.BlockSpec((c, hs), lambda t: (0, 0)),
        ],
        out_specs=pl.BlockSpec((tm, c), lambda t: (t, 0)),
        out_shape=jax.ShapeDtypeStruct((n, c), jnp.float32),
    )(x_flat, shared_gate_W, shared_down_W)


def _combine_body(sh_ref, r0_ref, r1_ref, w_ref, out_ref):
    wv = w_ref[...]
    out_ref[...] = (sh_ref[...] + wv[:, 0:1] * r0_ref[...]
                    + wv[:, 1:2] * r1_ref[...])


def _combine(shared_out, routed, w2):
    n, c = shared_out.shape
    tm = 512
    rt = n // tm
    return pl.pallas_call(
        _combine_body,
        grid=(rt,),
        in_specs=[
            pl.BlockSpec((tm, c), lambda t: (t, 0)),
            pl.BlockSpec((tm, c), lambda t: (t, 0)),
            pl.BlockSpec((tm, c), lambda t: (t + rt, 0)),
            pl.BlockSpec((tm, _TOPK), lambda t: (t, 0)),
        ],
        out_specs=pl.BlockSpec((tm, c), lambda t: (t, 0)),
        out_shape=jax.ShapeDtypeStruct((n, c), jnp.float32),
    )(shared_out, routed, routed, w2)


def kernel(x, gate_W, shared_gate_W, shared_down_W, expert_gate_W, expert_down_W):
    bs, ts, c = x.shape
    n = bs * ts
    x_flat = x.reshape(n, c)
    m = n * _TOPK
    mp = m + _E * _TM
    nt = mp // _TM

    w2, idx2, rnk2, tcnt = _gating(x_flat, gate_W)

    tcnt = tcnt.reshape(-1, _E)
    base_tile = jnp.cumsum(tcnt, axis=0) - tcnt
    cnt = jnp.sum(tcnt, axis=0)
    pad_cnt = ((cnt + _TM - 1) // _TM) * _TM
    pad_off = jnp.concatenate([jnp.zeros((1,), jnp.int32),
                               jnp.cumsum(pad_cnt).astype(jnp.int32)])
    tile_row = jnp.arange(nt, dtype=jnp.int32) * _TM
    tile_expert = jnp.clip(
        jnp.sum((tile_row[:, None] >= pad_off[None, 1:]).astype(jnp.int32),
                axis=1), 0, _E - 1).astype(jnp.int32)
    tile_used = (tile_row < pad_off[_E]).astype(jnp.int32)

    d0, d1 = _dst_rows(idx2, rnk2, base_tile, pad_off[:_E].reshape(1, _E))
    pos_all = jnp.concatenate([d0.reshape(n), d1.reshape(n)])

    xs = _sc_scatter_pairs(x_flat, pos_all, mp)
    shared_out = _shared_ffn(x_flat, shared_gate_W, shared_down_W)
    os_ = _grouped_ffn(xs, expert_gate_W, expert_down_W, tile_expert, tile_used)
    routed = _sc_gather(os_, pos_all)
    out = _combine(shared_out, routed, w2)
    return out.reshape(bs, ts, c)
